# Initial kernel scaffold; baseline (speedup 1.0000x reference)
import jax
import jax.numpy as jnp
from jax.experimental import pallas as pl


def kernel(x, edge_index, edge_weight, W, a):
    raise NotImplementedError("write your pallas kernel here")


# speedup vs baseline: 2.8070x; 2.8070x over previous
import functools

import jax
import jax.numpy as jnp
from jax import lax
from jax.experimental import pallas as pl
from jax.experimental.pallas import tpu as pltpu
from jax.experimental.pallas import tpu_sc as plsc

N_NODES = 10000
IN_DIM = 128
OUT_DIM = 128
N_EDGES = 320000

NC = 2
NS = 16
NW = NC * NS
LANES = 16
CHUNK = 128
K = (N_EDGES + NW * CHUNK - 1) // (NW * CHUNK)
E_PAD = NW * K * CHUNK
ROWS_PER_TILE = 624
SLAB = ((0, 128), (128, 128), (256, 128), (384, 128), (512, 112))


def _mm_body(x_ref, w_ref, o_ref):
    o_ref[...] = jnp.dot(x_ref[...], w_ref[...],
                         preferred_element_type=jnp.float32)


def _matmul(x, W):
    m = x.shape[0]
    bm = 1000
    return pl.pallas_call(
        _mm_body,
        grid=(m // bm,),
        in_specs=[
            pl.BlockSpec((bm, IN_DIM), lambda i: (i, 0)),
            pl.BlockSpec((IN_DIM, OUT_DIM), lambda i: (0, 0)),
        ],
        out_specs=pl.BlockSpec((bm, OUT_DIM), lambda i: (i, 0)),
        out_shape=jax.ShapeDtypeStruct((m, OUT_DIM), jnp.float32),
    )(x, W)


def _sc_body(xh_hbm, src_hbm, dst_hbm, ew_hbm, out_hbm,
             sidx_v, didx_v, ew_v, rows_v, acc, sem):
    cid = lax.axis_index("c")
    sid = lax.axis_index("s")
    wid = cid * NS + sid

    zero = jnp.zeros((LANES,), jnp.float32)

    def _zrow(r, _):
        for c in range(OUT_DIM // LANES):
            rows_v[r, pl.ds(c * LANES, LANES)] = zero
        return 0
    lax.fori_loop(0, CHUNK, _zrow, 0)
    base = sid * ROWS_PER_TILE
    for off, n in SLAB:
        pltpu.sync_copy(rows_v.at[pl.ds(0, n)], acc.at[pl.ds(base + off, n)])

    @pl.when(sid == NS - 1)
    def _zero_tail():
        pltpu.sync_copy(rows_v.at[pl.ds(0, N_NODES - NS * ROWS_PER_TILE)],
                        acc.at[pl.ds(NS * ROWS_PER_TILE,
                                     N_NODES - NS * ROWS_PER_TILE)])
    plsc.subcore_barrier()

    ebase = wid * K * CHUNK

    def _chunk(k, _):
        e0 = ebase + k * CHUNK
        pltpu.sync_copy(src_hbm.at[pl.ds(e0, CHUNK)], sidx_v)
        pltpu.sync_copy(dst_hbm.at[pl.ds(e0, CHUNK)], didx_v)
        pltpu.sync_copy(ew_hbm.at[pl.ds(e0 * LANES, CHUNK * LANES)], ew_v)
        pltpu.async_copy(xh_hbm.at[sidx_v], rows_v, sem).wait()

        def _row(r, _):
            w = ew_v[pl.ds(r * LANES, LANES)]
            for c in range(OUT_DIM // LANES):
                sl = pl.ds(c * LANES, LANES)
                rows_v[r, sl] = rows_v[r, sl] * w
            return 0
        lax.fori_loop(0, CHUNK, _row, 0)

        pltpu.sync_copy(rows_v, acc.at[didx_v], add=True)
        return 0
    lax.fori_loop(0, K, _chunk, 0)

    plsc.subcore_barrier()
    for off, n in SLAB:
        pltpu.sync_copy(acc.at[pl.ds(base + off, n)],
                        out_hbm.at[cid, pl.ds(base + off, n)])

    @pl.when(sid == NS - 1)
    def _write_tail():
        tail0 = NS * ROWS_PER_TILE
        ntail = N_NODES - tail0
        pltpu.sync_copy(acc.at[pl.ds(tail0, ntail)],
                        out_hbm.at[cid, pl.ds(tail0, ntail)])


_sc_call = pl.kernel(
    _sc_body,
    out_type=jax.ShapeDtypeStruct((NC, N_NODES, OUT_DIM), jnp.float32),
    mesh=plsc.VectorSubcoreMesh(core_axis_name="c", subcore_axis_name="s"),
    scratch_types=[
        pltpu.VMEM((CHUNK,), jnp.int32),
        pltpu.VMEM((CHUNK,), jnp.int32),
        pltpu.VMEM((CHUNK * LANES,), jnp.float32),
        pltpu.VMEM((CHUNK, OUT_DIM), jnp.float32),
        pltpu.VMEM_SHARED((N_NODES, OUT_DIM), jnp.float32),
        pltpu.SemaphoreType.DMA,
    ],
)


def _sc_edges(xh, src1, dst1, ew1):
    return _sc_call(xh, src1, dst1, ew1)


def _fin_body(a_ref, p_ref, o_ref):
    s = p_ref[0] + p_ref[1]
    slope = a_ref[0, 0]
    o_ref[...] = jnp.where(s > 0, s, slope * s)


def _finish(a2, partial):
    bm = 1000
    return pl.pallas_call(
        _fin_body,
        grid=(N_NODES // bm,),
        in_specs=[
            pl.BlockSpec(memory_space=pltpu.SMEM),
            pl.BlockSpec((NC, bm, OUT_DIM), lambda i: (0, i, 0)),
        ],
        out_specs=pl.BlockSpec((bm, OUT_DIM), lambda i: (i, 0)),
        out_shape=jax.ShapeDtypeStruct((N_NODES, OUT_DIM), jnp.float32),
    )(a2, partial)


@jax.jit
def kernel(x, edge_index, edge_weight, W, a):
    xh = _matmul(x, W)

    dst = edge_index[0].astype(jnp.int32)
    src = edge_index[1].astype(jnp.int32)
    ew = edge_weight.astype(jnp.float32)
    pad = E_PAD - N_EDGES
    src1 = jnp.pad(src, (0, pad))
    dst1 = jnp.pad(dst, (0, pad))
    ew1 = jnp.pad(ew, (0, pad))
    ew16 = jnp.broadcast_to(ew1[:, None], (E_PAD, LANES)).reshape(-1)

    partial = _sc_edges(xh, src1, dst1, ew16)

    a2 = jnp.reshape(a, (1, 1)).astype(jnp.float32)
    return _finish(a2, partial)
